# baseline (speedup 1.0000x reference)
import functools

import jax
import jax.numpy as jnp
from jax import lax
from jax.experimental import pallas as pl
from jax.experimental.pallas import tpu as pltpu
from jax.experimental.pallas import tpu_sc as plsc

N = 10000
NPAD = 10240
E = 320000
NC, NS = 2, 16
NW = NC * NS
CH = 64
SCK = 18
NSC = 9
NITER = SCK * NSC
EW = NITER * CH
EP = NW * EW
TR = NPAD // NS
ZR = 128

_MESH = plsc.VectorSubcoreMesh(
    core_axis_name="c", subcore_axis_name="s", num_cores=NC, num_subcores=NS
)


def _make_sc_scatter(C):
  @functools.partial(
      pl.kernel,
      out_type=jax.ShapeDtypeStruct((NC, NPAD, C), jnp.float32),
      mesh=_MESH,
      compiler_params=pltpu.CompilerParams(use_tc_tiling_on_sc=False),
      scratch_types=[
          pltpu.VMEM((SCK, CH), jnp.int32),
          pltpu.VMEM((SCK, CH), jnp.int32),
          pltpu.VMEM((CH, C), jnp.float32),
          pltpu.VMEM((CH, C), jnp.float32),
          pltpu.VMEM((CH, C), jnp.float32),
          pltpu.VMEM_SHARED((NPAD, C), jnp.float32),
          pltpu.SemaphoreType.DMA,
          pltpu.SemaphoreType.DMA,
          pltpu.SemaphoreType.DMA,
          pltpu.SemaphoreType.DMA,
          pltpu.SemaphoreType.DMA,
          pltpu.SemaphoreType.DMA,
      ],
  )
  def sc_scatter(src_hbm, dst_hbm, u_hbm, z_hbm, out_hbm,
                 srcs_v, dsts_v, rowsA, rowsB, rowsC, acc_sh,
                 gsA, gsB, gsC, ssA, ssB, ssC):
    c = lax.axis_index("c")
    s = lax.axis_index("s")
    wid = c * NS + s

    def g(i, rows, sem):
      pltpu.async_copy(u_hbm.at[srcs_v.at[i]], rows, sem)

    def gwait(i, rows, sem):
      pltpu.make_async_copy(u_hbm.at[srcs_v.at[i]], rows, sem).wait()

    def sct(i, rows, sem):
      pltpu.async_copy(rows, acc_sh.at[dsts_v.at[i]], sem, add=True)

    def swait(i, rows, sem):
      pltpu.make_async_copy(rows, acc_sh.at[dsts_v.at[i]], sem).wait()

    @pl.loop(0, TR // ZR)
    def _(i):
      pltpu.sync_copy(z_hbm, acc_sh.at[pl.ds(s * TR + i * ZR, ZR)])

    plsc.subcore_barrier()

    @pl.loop(0, NSC)
    def _(sc):
      pltpu.sync_copy(src_hbm.at[wid, pl.ds(sc * SCK, SCK)], srcs_v)
      pltpu.sync_copy(dst_hbm.at[wid, pl.ds(sc * SCK, SCK)], dsts_v)

      g(0, rowsA, gsA)
      g(1, rowsB, gsB)
      gwait(0, rowsA, gsA)
      sct(0, rowsA, ssA)
      g(2, rowsC, gsC)

      @pl.loop(0, SCK // 3 - 1)
      def _(t):
        i = 3 * t
        gwait(i + 1, rowsB, gsB)
        sct(i + 1, rowsB, ssB)
        swait(i, rowsA, ssA)
        g(i + 3, rowsA, gsA)
        gwait(i + 2, rowsC, gsC)
        sct(i + 2, rowsC, ssC)
        swait(i + 1, rowsB, ssB)
        g(i + 4, rowsB, gsB)
        gwait(i + 3, rowsA, gsA)
        sct(i + 3, rowsA, ssA)
        swait(i + 2, rowsC, ssC)
        g(i + 5, rowsC, gsC)

      gwait(SCK - 2, rowsB, gsB)
      sct(SCK - 2, rowsB, ssB)
      gwait(SCK - 1, rowsC, gsC)
      sct(SCK - 1, rowsC, ssC)
      swait(SCK - 3, rowsA, ssA)
      swait(SCK - 2, rowsB, ssB)
      swait(SCK - 1, rowsC, ssC)

    plsc.subcore_barrier()
    pltpu.sync_copy(acc_sh.at[pl.ds(s * TR, TR)],
                    out_hbm.at[c, pl.ds(s * TR, TR)])

  return sc_scatter


def _make_sc_degree(C):
  @functools.partial(
      pl.kernel,
      out_type=jax.ShapeDtypeStruct((NC, NPAD, C), jnp.float32),
      mesh=_MESH,
      compiler_params=pltpu.CompilerParams(use_tc_tiling_on_sc=False),
      scratch_types=[
          pltpu.VMEM((SCK, CH), jnp.int32),
          pltpu.VMEM((CH, C), jnp.float32),
          pltpu.VMEM_SHARED((NPAD, C), jnp.float32),
          pltpu.SemaphoreType.DMA,
      ],
  )
  def sc_degree(dst_hbm, ones_hbm, z_hbm, out_hbm,
                dsts_v, ones_v, acc_sh, sem):
    c = lax.axis_index("c")
    s = lax.axis_index("s")
    wid = c * NS + s

    pltpu.sync_copy(ones_hbm, ones_v)

    @pl.loop(0, TR // ZR)
    def _(i):
      pltpu.sync_copy(z_hbm, acc_sh.at[pl.ds(s * TR + i * ZR, ZR)])

    plsc.subcore_barrier()

    @pl.loop(0, NSC)
    def _(sc):
      pltpu.sync_copy(dst_hbm.at[wid, pl.ds(sc * SCK, SCK)], dsts_v)

      @pl.loop(0, SCK)
      def _(i):
        pltpu.sync_copy(ones_v, acc_sh.at[dsts_v.at[i]], add=True)

    plsc.subcore_barrier()
    pltpu.sync_copy(acc_sh.at[pl.ds(s * TR, TR)],
                    out_hbm.at[c, pl.ds(s * TR, TR)])

  return sc_degree


_sc_deg = _make_sc_degree(8)
_sc_scatter128 = _make_sc_scatter(128)
_sc_scatter48 = _make_sc_scatter(48)

_BLK = 1000
_GRID = N // _BLK


def _tc_norm_body(d0_ref, d1_ref, x_ref, dinv_ref, u1_ref):
  deg = d0_ref[0] + d1_ref[0] + 1.0
  dinv = lax.rsqrt(deg)
  dinv_ref[...] = dinv
  u1_ref[...] = dinv[:, 0:1] * x_ref[...]


def _tc_norm(degs, x):
  return pl.pallas_call(
      _tc_norm_body,
      grid=(_GRID,),
      in_specs=[
          pl.BlockSpec((1, _BLK, 8), lambda i: (0, i, 0)),
          pl.BlockSpec((1, _BLK, 8), lambda i: (1, i, 0)),
          pl.BlockSpec((_BLK, 128), lambda i: (i, 0)),
      ],
      out_specs=[
          pl.BlockSpec((_BLK, 8), lambda i: (i, 0)),
          pl.BlockSpec((_BLK, 128), lambda i: (i, 0)),
      ],
      out_shape=[
          jax.ShapeDtypeStruct((N, 8), jnp.float32),
          jax.ShapeDtypeStruct((N, 128), jnp.float32),
      ],
  )(degs, degs, x)


def _tc_mid_body(a0_ref, a1_ref, u1_ref, dinv_ref, w1_ref, b1_ref, w2_ref,
                 u2_ref):
  dv = dinv_ref[:, 0:1]
  y1 = dv * (a0_ref[0] + a1_ref[0] + u1_ref[...])
  h = jnp.dot(y1, w1_ref[...], preferred_element_type=jnp.float32,
              precision=lax.Precision.HIGHEST)
  h = jnp.maximum(h + b1_ref[...], 0.0)
  g = jnp.dot(h, w2_ref[...], preferred_element_type=jnp.float32,
              precision=lax.Precision.HIGHEST)
  u2_ref[...] = dv * g


def _tc_mid(acc1, u1, dinv, W1, b1, W2p):
  return pl.pallas_call(
      _tc_mid_body,
      grid=(_GRID,),
      in_specs=[
          pl.BlockSpec((1, _BLK, 128), lambda i: (0, i, 0)),
          pl.BlockSpec((1, _BLK, 128), lambda i: (1, i, 0)),
          pl.BlockSpec((_BLK, 128), lambda i: (i, 0)),
          pl.BlockSpec((_BLK, 8), lambda i: (i, 0)),
          pl.BlockSpec((128, 256), lambda i: (0, 0)),
          pl.BlockSpec((1, 256), lambda i: (0, 0)),
          pl.BlockSpec((256, 48), lambda i: (0, 0)),
      ],
      out_specs=pl.BlockSpec((_BLK, 48), lambda i: (i, 0)),
      out_shape=jax.ShapeDtypeStruct((N, 48), jnp.float32),
  )(acc1, acc1, u1, dinv, W1, b1, W2p)


def _tc_out_body(a0_ref, a1_ref, u2_ref, dinv_ref, b2_ref, out_ref):
  y = dinv_ref[:, 0:1] * (a0_ref[0] + a1_ref[0] + u2_ref[...])
  y = y[:, 0:40] + b2_ref[...]
  m = jnp.max(y, axis=1, keepdims=True)
  ys = y - m
  out_ref[...] = ys - jnp.log(jnp.sum(jnp.exp(ys), axis=1, keepdims=True))


def _tc_out(acc2, u2, dinv, b2):
  return pl.pallas_call(
      _tc_out_body,
      grid=(_GRID,),
      in_specs=[
          pl.BlockSpec((1, _BLK, 48), lambda i: (0, i, 0)),
          pl.BlockSpec((1, _BLK, 48), lambda i: (1, i, 0)),
          pl.BlockSpec((_BLK, 48), lambda i: (i, 0)),
          pl.BlockSpec((_BLK, 8), lambda i: (i, 0)),
          pl.BlockSpec((1, 40), lambda i: (0, 0)),
      ],
      out_specs=pl.BlockSpec((_BLK, 40), lambda i: (i, 0)),
      out_shape=jax.ShapeDtypeStruct((N, 40), jnp.float32),
  )(acc2, acc2, u2, dinv, b2)


def kernel(x, edge_index, W1, b1, W2, b2):
  src = jnp.concatenate(
      [edge_index[0], jnp.zeros((EP - E,), jnp.int32)]).reshape(NW, NITER, CH)
  dst = jnp.concatenate(
      [edge_index[1],
       jnp.full((EP - E,), NPAD - 1, jnp.int32)]).reshape(NW, NITER, CH)

  ones8 = jnp.ones((CH, 8), jnp.float32)
  z8 = jnp.zeros((ZR, 8), jnp.float32)
  z128 = jnp.zeros((ZR, 128), jnp.float32)
  z48 = jnp.zeros((ZR, 48), jnp.float32)
  W2p = jnp.pad(W2, ((0, 0), (0, 8)))
  b1r = b1.reshape(1, 256)
  b2r = b2.reshape(1, 40)

  degs = _sc_deg(dst, ones8, z8)
  dinv, u1 = _tc_norm(degs, x)
  acc1 = _sc_scatter128(src, dst, u1, z128)
  u2 = _tc_mid(acc1, u1, dinv, W1, b1r, W2p)
  acc2 = _sc_scatter48(src, dst, u2, z48)
  return _tc_out(acc2, u2, dinv, b2r)


# speedup vs baseline: 2.0736x; 2.0736x over previous
import functools

import jax
import jax.numpy as jnp
from jax import lax
from jax.experimental import pallas as pl
from jax.experimental.pallas import tpu as pltpu
from jax.experimental.pallas import tpu_sc as plsc

N = 10000
NPAD = 10240
E = 320000
NC, NS = 2, 16
NW = NC * NS
CH = 80
SCK = 25
NSC = 5
NITER = SCK * NSC
EW = NITER * CH
EP = NW * EW
TR = NPAD // NS
ZR = 128

_MESH = plsc.VectorSubcoreMesh(
    core_axis_name="c", subcore_axis_name="s", num_cores=NC, num_subcores=NS
)


def _make_sc_scatter(C):
  @functools.partial(
      pl.kernel,
      out_type=jax.ShapeDtypeStruct((NC, NPAD, C), jnp.float32),
      mesh=_MESH,
      compiler_params=pltpu.CompilerParams(use_tc_tiling_on_sc=False),
      scratch_types=[
          pltpu.VMEM((SCK, CH), jnp.int32),
          pltpu.VMEM((SCK, CH), jnp.int32),
          pltpu.VMEM((CH, C), jnp.float32),
          pltpu.VMEM((CH, C), jnp.float32),
          pltpu.VMEM_SHARED((NPAD, C), jnp.float32),
          pltpu.SemaphoreType.DMA,
          pltpu.SemaphoreType.DMA,
      ],
  )
  def sc_scatter(src_hbm, dst_hbm, u_hbm, z_hbm, out_hbm,
                 srcs_v, dsts_v, rows0, rows1, acc_sh, gsem0, gsem1):
    c = lax.axis_index("c")
    s = lax.axis_index("s")
    wid = c * NS + s

    @pl.loop(0, TR // ZR)
    def _(i):
      pltpu.sync_copy(z_hbm, acc_sh.at[pl.ds(s * TR + i * ZR, ZR)])

    plsc.subcore_barrier()

    @pl.loop(0, NSC)
    def _(sc):
      pltpu.sync_copy(src_hbm.at[wid, pl.ds(sc * SCK, SCK)], srcs_v)
      pltpu.sync_copy(dst_hbm.at[wid, pl.ds(sc * SCK, SCK)], dsts_v)
      pltpu.async_copy(u_hbm.at[srcs_v.at[0]], rows0, gsem0)

      @pl.loop(0, (SCK - 1) // 2)
      def _(j):
        i0 = 2 * j
        pltpu.make_async_copy(u_hbm.at[srcs_v.at[i0]], rows0, gsem0).wait()
        pltpu.async_copy(u_hbm.at[srcs_v.at[i0 + 1]], rows1, gsem1)
        pltpu.sync_copy(rows0, acc_sh.at[dsts_v.at[i0]], add=True)
        pltpu.make_async_copy(u_hbm.at[srcs_v.at[i0 + 1]], rows1, gsem1).wait()
        pltpu.async_copy(u_hbm.at[srcs_v.at[i0 + 2]], rows0, gsem0)
        pltpu.sync_copy(rows1, acc_sh.at[dsts_v.at[i0 + 1]], add=True)

      pltpu.make_async_copy(u_hbm.at[srcs_v.at[SCK - 1]], rows0, gsem0).wait()
      pltpu.sync_copy(rows0, acc_sh.at[dsts_v.at[SCK - 1]], add=True)

    plsc.subcore_barrier()
    pltpu.sync_copy(acc_sh.at[pl.ds(s * TR, TR)],
                    out_hbm.at[c, pl.ds(s * TR, TR)])

  return sc_scatter


def _make_sc_degree(C):
  @functools.partial(
      pl.kernel,
      out_type=jax.ShapeDtypeStruct((NC, NPAD, C), jnp.float32),
      mesh=_MESH,
      compiler_params=pltpu.CompilerParams(use_tc_tiling_on_sc=False),
      scratch_types=[
          pltpu.VMEM((SCK, CH), jnp.int32),
          pltpu.VMEM((CH, C), jnp.float32),
          pltpu.VMEM_SHARED((NPAD, C), jnp.float32),
          pltpu.SemaphoreType.DMA,
      ],
  )
  def sc_degree(dst_hbm, ones_hbm, z_hbm, out_hbm,
                dsts_v, ones_v, acc_sh, sem):
    c = lax.axis_index("c")
    s = lax.axis_index("s")
    wid = c * NS + s

    pltpu.sync_copy(ones_hbm, ones_v)

    @pl.loop(0, TR // ZR)
    def _(i):
      pltpu.sync_copy(z_hbm, acc_sh.at[pl.ds(s * TR + i * ZR, ZR)])

    plsc.subcore_barrier()

    @pl.loop(0, NSC)
    def _(sc):
      pltpu.sync_copy(dst_hbm.at[wid, pl.ds(sc * SCK, SCK)], dsts_v)

      @pl.loop(0, SCK)
      def _(i):
        pltpu.sync_copy(ones_v, acc_sh.at[dsts_v.at[i]], add=True)

    plsc.subcore_barrier()
    pltpu.sync_copy(acc_sh.at[pl.ds(s * TR, TR)],
                    out_hbm.at[c, pl.ds(s * TR, TR)])

  return sc_degree


_sc_deg = _make_sc_degree(8)
_sc_scatter128 = _make_sc_scatter(128)
_sc_scatter48 = _make_sc_scatter(48)

_BLK = 1000
_GRID = N // _BLK


def _tc_norm_body(d0_ref, d1_ref, x_ref, dinv_ref, u1_ref):
  deg = d0_ref[0] + d1_ref[0] + 1.0
  dinv = lax.rsqrt(deg)
  dinv_ref[...] = dinv
  u1_ref[...] = dinv[:, 0:1] * x_ref[...]


def _tc_norm(degs, x):
  return pl.pallas_call(
      _tc_norm_body,
      grid=(_GRID,),
      in_specs=[
          pl.BlockSpec((1, _BLK, 8), lambda i: (0, i, 0)),
          pl.BlockSpec((1, _BLK, 8), lambda i: (1, i, 0)),
          pl.BlockSpec((_BLK, 128), lambda i: (i, 0)),
      ],
      out_specs=[
          pl.BlockSpec((_BLK, 8), lambda i: (i, 0)),
          pl.BlockSpec((_BLK, 128), lambda i: (i, 0)),
      ],
      out_shape=[
          jax.ShapeDtypeStruct((N, 8), jnp.float32),
          jax.ShapeDtypeStruct((N, 128), jnp.float32),
      ],
  )(degs, degs, x)


def _tc_mid_body(a0_ref, a1_ref, u1_ref, dinv_ref, w1_ref, b1_ref, w2_ref,
                 u2_ref):
  dv = dinv_ref[:, 0:1]
  y1 = dv * (a0_ref[0] + a1_ref[0] + u1_ref[...])
  h = jnp.dot(y1, w1_ref[...], preferred_element_type=jnp.float32,
              precision=lax.Precision.HIGHEST)
  h = jnp.maximum(h + b1_ref[...], 0.0)
  g = jnp.dot(h, w2_ref[...], preferred_element_type=jnp.float32,
              precision=lax.Precision.HIGHEST)
  u2_ref[...] = dv * g


def _tc_mid(acc1, u1, dinv, W1, b1, W2p):
  return pl.pallas_call(
      _tc_mid_body,
      grid=(_GRID,),
      in_specs=[
          pl.BlockSpec((1, _BLK, 128), lambda i: (0, i, 0)),
          pl.BlockSpec((1, _BLK, 128), lambda i: (1, i, 0)),
          pl.BlockSpec((_BLK, 128), lambda i: (i, 0)),
          pl.BlockSpec((_BLK, 8), lambda i: (i, 0)),
          pl.BlockSpec((128, 256), lambda i: (0, 0)),
          pl.BlockSpec((1, 256), lambda i: (0, 0)),
          pl.BlockSpec((256, 48), lambda i: (0, 0)),
      ],
      out_specs=pl.BlockSpec((_BLK, 48), lambda i: (i, 0)),
      out_shape=jax.ShapeDtypeStruct((N, 48), jnp.float32),
  )(acc1, acc1, u1, dinv, W1, b1, W2p)


def _tc_out_body(a0_ref, a1_ref, u2_ref, dinv_ref, b2_ref, out_ref):
  y = dinv_ref[:, 0:1] * (a0_ref[0] + a1_ref[0] + u2_ref[...])
  y = y[:, 0:40] + b2_ref[...]
  m = jnp.max(y, axis=1, keepdims=True)
  ys = y - m
  out_ref[...] = ys - jnp.log(jnp.sum(jnp.exp(ys), axis=1, keepdims=True))


def _tc_out(acc2, u2, dinv, b2):
  return pl.pallas_call(
      _tc_out_body,
      grid=(_GRID,),
      in_specs=[
          pl.BlockSpec((1, _BLK, 48), lambda i: (0, i, 0)),
          pl.BlockSpec((1, _BLK, 48), lambda i: (1, i, 0)),
          pl.BlockSpec((_BLK, 48), lambda i: (i, 0)),
          pl.BlockSpec((_BLK, 8), lambda i: (i, 0)),
          pl.BlockSpec((1, 40), lambda i: (0, 0)),
      ],
      out_specs=pl.BlockSpec((_BLK, 40), lambda i: (i, 0)),
      out_shape=jax.ShapeDtypeStruct((N, 40), jnp.float32),
  )(acc2, acc2, u2, dinv, b2)


def kernel(x, edge_index, W1, b1, W2, b2):
  src = jnp.concatenate(
      [edge_index[0], jnp.zeros((EP - E,), jnp.int32)]).reshape(NW, NITER, CH)
  dst = jnp.concatenate(
      [edge_index[1],
       jnp.full((EP - E,), NPAD - 1, jnp.int32)]).reshape(NW, NITER, CH)

  ones8 = jnp.ones((CH, 8), jnp.float32)
  z8 = jnp.zeros((ZR, 8), jnp.float32)
  z128 = jnp.zeros((ZR, 128), jnp.float32)
  z48 = jnp.zeros((ZR, 48), jnp.float32)
  W2p = jnp.pad(W2, ((0, 0), (0, 8)))
  b1r = b1.reshape(1, 256)
  b2r = b2.reshape(1, 40)

  degs = _sc_deg(dst, ones8, z8)
  dinv, u1 = _tc_norm(degs, x)
  acc1 = _sc_scatter128(src, dst, u1, z128)
  u2 = _tc_mid(acc1, u1, dinv, W1, b1r, W2p)
  acc2 = _sc_scatter48(src, dst, u2, z48)
  return _tc_out(acc2, u2, dinv, b2r)


# speedup vs baseline: 2.1852x; 1.0539x over previous
import functools

import jax
import jax.numpy as jnp
from jax import lax
from jax.experimental import pallas as pl
from jax.experimental.pallas import tpu as pltpu
from jax.experimental.pallas import tpu_sc as plsc

N = 10000
NPAD = 10240
E = 320000
NC, NS = 2, 16
NW = NC * NS
CH = 80
SCK = 25
NSC = 5
NITER = SCK * NSC
EW = NITER * CH
EP = NW * EW
TR = NPAD // NS
ZR = 128

_MESH = plsc.VectorSubcoreMesh(
    core_axis_name="c", subcore_axis_name="s", num_cores=NC, num_subcores=NS
)


def _make_sc_scatter(C, tc_tiling):
  @functools.partial(
      pl.kernel,
      out_type=jax.ShapeDtypeStruct((NC, NPAD, C), jnp.float32),
      mesh=_MESH,
      compiler_params=pltpu.CompilerParams(use_tc_tiling_on_sc=tc_tiling),
      scratch_types=[
          pltpu.VMEM((SCK, CH), jnp.int32),
          pltpu.VMEM((SCK, CH), jnp.int32),
          pltpu.VMEM((CH, C), jnp.float32),
          pltpu.VMEM((CH, C), jnp.float32),
          pltpu.VMEM_SHARED((NPAD, C), jnp.float32),
          pltpu.SemaphoreType.DMA,
          pltpu.SemaphoreType.DMA,
      ],
  )
  def sc_scatter(src_hbm, dst_hbm, u_hbm, z_hbm, out_hbm,
                 srcs_v, dsts_v, rows0, rows1, acc_sh, gsem0, gsem1):
    c = lax.axis_index("c")
    s = lax.axis_index("s")
    wid = c * NS + s

    @pl.loop(0, TR // ZR)
    def _(i):
      pltpu.sync_copy(z_hbm, acc_sh.at[pl.ds(s * TR + i * ZR, ZR)])

    plsc.subcore_barrier()

    @pl.loop(0, NSC)
    def _(sc):
      pltpu.sync_copy(src_hbm.at[wid, sc], srcs_v)
      pltpu.sync_copy(dst_hbm.at[wid, sc], dsts_v)
      pltpu.async_copy(u_hbm.at[srcs_v.at[0]], rows0, gsem0)

      @pl.loop(0, (SCK - 1) // 2)
      def _(j):
        i0 = 2 * j
        pltpu.make_async_copy(u_hbm.at[srcs_v.at[i0]], rows0, gsem0).wait()
        pltpu.async_copy(u_hbm.at[srcs_v.at[i0 + 1]], rows1, gsem1)
        pltpu.sync_copy(rows0, acc_sh.at[dsts_v.at[i0]], add=True)
        pltpu.make_async_copy(u_hbm.at[srcs_v.at[i0 + 1]], rows1, gsem1).wait()
        pltpu.async_copy(u_hbm.at[srcs_v.at[i0 + 2]], rows0, gsem0)
        pltpu.sync_copy(rows1, acc_sh.at[dsts_v.at[i0 + 1]], add=True)

      pltpu.make_async_copy(u_hbm.at[srcs_v.at[SCK - 1]], rows0, gsem0).wait()
      pltpu.sync_copy(rows0, acc_sh.at[dsts_v.at[SCK - 1]], add=True)

    plsc.subcore_barrier()
    pltpu.sync_copy(acc_sh.at[pl.ds(s * TR, TR)],
                    out_hbm.at[c, pl.ds(s * TR, TR)])

  return sc_scatter


def _make_sc_degree(C, tc_tiling):
  @functools.partial(
      pl.kernel,
      out_type=jax.ShapeDtypeStruct((NC, NPAD, C), jnp.float32),
      mesh=_MESH,
      compiler_params=pltpu.CompilerParams(use_tc_tiling_on_sc=tc_tiling),
      scratch_types=[
          pltpu.VMEM((SCK, CH), jnp.int32),
          pltpu.VMEM((CH, C), jnp.float32),
          pltpu.VMEM_SHARED((NPAD, C), jnp.float32),
          pltpu.SemaphoreType.DMA,
      ],
  )
  def sc_degree(dst_hbm, ones_hbm, z_hbm, out_hbm,
                dsts_v, ones_v, acc_sh, sem):
    c = lax.axis_index("c")
    s = lax.axis_index("s")
    wid = c * NS + s

    pltpu.sync_copy(ones_hbm, ones_v)

    @pl.loop(0, TR // ZR)
    def _(i):
      pltpu.sync_copy(z_hbm, acc_sh.at[pl.ds(s * TR + i * ZR, ZR)])

    plsc.subcore_barrier()

    @pl.loop(0, NSC)
    def _(sc):
      pltpu.sync_copy(dst_hbm.at[wid, sc], dsts_v)

      @pl.loop(0, SCK)
      def _(i):
        pltpu.sync_copy(ones_v, acc_sh.at[dsts_v.at[i]], add=True)

    plsc.subcore_barrier()
    pltpu.sync_copy(acc_sh.at[pl.ds(s * TR, TR)],
                    out_hbm.at[c, pl.ds(s * TR, TR)])

  return sc_degree


_sc_deg = _make_sc_degree(8, False)
_sc_scatter128 = _make_sc_scatter(128, True)
_sc_scatter48 = _make_sc_scatter(48, False)

_BLK = 1000
_GRID = N // _BLK


def _tc_norm_body(d0_ref, d1_ref, x_ref, dinv_ref, u1_ref):
  deg = d0_ref[0] + d1_ref[0] + 1.0
  dinv = lax.rsqrt(deg)
  dinv_ref[...] = dinv
  u1_ref[...] = dinv[:, 0:1] * x_ref[...]


def _tc_norm(degs, x):
  return pl.pallas_call(
      _tc_norm_body,
      grid=(_GRID,),
      in_specs=[
          pl.BlockSpec((1, _BLK, 8), lambda i: (0, i, 0)),
          pl.BlockSpec((1, _BLK, 8), lambda i: (1, i, 0)),
          pl.BlockSpec((_BLK, 128), lambda i: (i, 0)),
      ],
      out_specs=[
          pl.BlockSpec((_BLK, 8), lambda i: (i, 0)),
          pl.BlockSpec((_BLK, 128), lambda i: (i, 0)),
      ],
      out_shape=[
          jax.ShapeDtypeStruct((N, 8), jnp.float32),
          jax.ShapeDtypeStruct((N, 128), jnp.float32),
      ],
  )(degs, degs, x)


def _tc_mid_body(a0_ref, a1_ref, u1_ref, dinv_ref, w1_ref, b1_ref, w2_ref,
                 u2_ref):
  dv = dinv_ref[:, 0:1]
  y1 = dv * (a0_ref[0] + a1_ref[0] + u1_ref[...])
  h = jnp.dot(y1, w1_ref[...], preferred_element_type=jnp.float32)
  h = jnp.maximum(h + b1_ref[...], 0.0)
  g = jnp.dot(h, w2_ref[...], preferred_element_type=jnp.float32)
  u2_ref[...] = dv * g


def _tc_mid(acc1, u1, dinv, W1, b1, W2p):
  return pl.pallas_call(
      _tc_mid_body,
      grid=(_GRID,),
      in_specs=[
          pl.BlockSpec((1, _BLK, 128), lambda i: (0, i, 0)),
          pl.BlockSpec((1, _BLK, 128), lambda i: (1, i, 0)),
          pl.BlockSpec((_BLK, 128), lambda i: (i, 0)),
          pl.BlockSpec((_BLK, 8), lambda i: (i, 0)),
          pl.BlockSpec((128, 256), lambda i: (0, 0)),
          pl.BlockSpec((1, 256), lambda i: (0, 0)),
          pl.BlockSpec((256, 48), lambda i: (0, 0)),
      ],
      out_specs=pl.BlockSpec((_BLK, 48), lambda i: (i, 0)),
      out_shape=jax.ShapeDtypeStruct((N, 48), jnp.float32),
  )(acc1, acc1, u1, dinv, W1, b1, W2p)


def _tc_out_body(a0_ref, a1_ref, u2_ref, dinv_ref, b2_ref, out_ref):
  y = dinv_ref[:, 0:1] * (a0_ref[0] + a1_ref[0] + u2_ref[...])
  y = y[:, 0:40] + b2_ref[...]
  m = jnp.max(y, axis=1, keepdims=True)
  ys = y - m
  out_ref[...] = ys - jnp.log(jnp.sum(jnp.exp(ys), axis=1, keepdims=True))


def _tc_out(acc2, u2, dinv, b2):
  return pl.pallas_call(
      _tc_out_body,
      grid=(_GRID,),
      in_specs=[
          pl.BlockSpec((1, _BLK, 48), lambda i: (0, i, 0)),
          pl.BlockSpec((1, _BLK, 48), lambda i: (1, i, 0)),
          pl.BlockSpec((_BLK, 48), lambda i: (i, 0)),
          pl.BlockSpec((_BLK, 8), lambda i: (i, 0)),
          pl.BlockSpec((1, 40), lambda i: (0, 0)),
      ],
      out_specs=pl.BlockSpec((_BLK, 40), lambda i: (i, 0)),
      out_shape=jax.ShapeDtypeStruct((N, 40), jnp.float32),
  )(acc2, acc2, u2, dinv, b2)


def kernel(x, edge_index, W1, b1, W2, b2):
  src = jnp.concatenate(
      [edge_index[0],
       jnp.zeros((EP - E,), jnp.int32)]).reshape(NW, NSC, SCK, CH)
  dst = jnp.concatenate(
      [edge_index[1],
       jnp.full((EP - E,), NPAD - 1, jnp.int32)]).reshape(NW, NSC, SCK, CH)

  ones8 = jnp.ones((CH, 8), jnp.float32)
  z8 = jnp.zeros((ZR, 8), jnp.float32)
  z128 = jnp.zeros((ZR, 128), jnp.float32)
  z48 = jnp.zeros((ZR, 48), jnp.float32)
  W2p = jnp.pad(W2, ((0, 0), (0, 8)))
  b1r = b1.reshape(1, 256)
  b2r = b2.reshape(1, 40)

  degs = _sc_deg(dst, ones8, z8)
  dinv, u1 = _tc_norm(degs, x)
  acc1 = _sc_scatter128(src, dst, u1, z128)
  u2 = _tc_mid(acc1, u1, dinv, W1, b1r, W2p)
  acc2 = _sc_scatter48(src, dst, u2, z48)
  return _tc_out(acc2, u2, dinv, b2r)


# speedup vs baseline: 2.2140x; 1.0132x over previous
import functools

import jax
import jax.numpy as jnp
from jax import lax
from jax.experimental import pallas as pl
from jax.experimental.pallas import tpu as pltpu
from jax.experimental.pallas import tpu_sc as plsc

N = 10000
NPAD = 10240
E = 320000
NC, NS = 2, 16
NW = NC * NS
CH = 80
SCK = 25
NSC = 5
NITER = SCK * NSC
EW = NITER * CH
EP = NW * EW
TR = NPAD // NS
ZR = 128

_MESH = plsc.VectorSubcoreMesh(
    core_axis_name="c", subcore_axis_name="s", num_cores=NC, num_subcores=NS
)


def _make_sc_scatter(C, tc_tiling):
  @functools.partial(
      pl.kernel,
      out_type=jax.ShapeDtypeStruct((NC, NPAD, C), jnp.float32),
      mesh=_MESH,
      compiler_params=pltpu.CompilerParams(use_tc_tiling_on_sc=tc_tiling),
      scratch_types=[
          pltpu.VMEM((SCK, CH), jnp.int32),
          pltpu.VMEM((SCK, CH), jnp.int32),
          pltpu.VMEM((CH, C), jnp.float32),
          pltpu.VMEM((CH, C), jnp.float32),
          pltpu.VMEM_SHARED((NPAD, C), jnp.float32),
          pltpu.SemaphoreType.DMA,
          pltpu.SemaphoreType.DMA,
      ],
  )
  def sc_scatter(src_hbm, dst_hbm, u_hbm, z_hbm, out_hbm,
                 srcs_v, dsts_v, rows0, rows1, acc_sh, gsem0, gsem1):
    c = lax.axis_index("c")
    s = lax.axis_index("s")
    wid = c * NS + s

    @pl.loop(0, TR // ZR)
    def _(i):
      pltpu.sync_copy(z_hbm, acc_sh.at[pl.ds(s * TR + i * ZR, ZR)])

    plsc.subcore_barrier()

    @pl.loop(0, NSC)
    def _(sc):
      pltpu.sync_copy(src_hbm.at[wid, sc], srcs_v)
      pltpu.sync_copy(dst_hbm.at[wid, sc], dsts_v)
      pltpu.async_copy(u_hbm.at[srcs_v.at[0]], rows0, gsem0)

      @pl.loop(0, (SCK - 1) // 2)
      def _(j):
        i0 = 2 * j
        pltpu.make_async_copy(u_hbm.at[srcs_v.at[i0]], rows0, gsem0).wait()
        pltpu.async_copy(u_hbm.at[srcs_v.at[i0 + 1]], rows1, gsem1)
        pltpu.sync_copy(rows0, acc_sh.at[dsts_v.at[i0]], add=True)
        pltpu.make_async_copy(u_hbm.at[srcs_v.at[i0 + 1]], rows1, gsem1).wait()
        pltpu.async_copy(u_hbm.at[srcs_v.at[i0 + 2]], rows0, gsem0)
        pltpu.sync_copy(rows1, acc_sh.at[dsts_v.at[i0 + 1]], add=True)

      pltpu.make_async_copy(u_hbm.at[srcs_v.at[SCK - 1]], rows0, gsem0).wait()
      pltpu.sync_copy(rows0, acc_sh.at[dsts_v.at[SCK - 1]], add=True)

    plsc.subcore_barrier()
    pltpu.sync_copy(acc_sh.at[pl.ds(s * TR, TR)],
                    out_hbm.at[c, pl.ds(s * TR, TR)])

  return sc_scatter


def _make_sc_degree(C, tc_tiling):
  @functools.partial(
      pl.kernel,
      out_type=jax.ShapeDtypeStruct((NC, NPAD, C), jnp.float32),
      mesh=_MESH,
      compiler_params=pltpu.CompilerParams(use_tc_tiling_on_sc=tc_tiling),
      scratch_types=[
          pltpu.VMEM((SCK, CH), jnp.int32),
          pltpu.VMEM((CH, C), jnp.float32),
          pltpu.VMEM_SHARED((NPAD, C), jnp.float32),
          pltpu.SemaphoreType.DMA,
      ],
  )
  def sc_degree(dst_hbm, ones_hbm, z_hbm, out_hbm,
                dsts_v, ones_v, acc_sh, sem):
    c = lax.axis_index("c")
    s = lax.axis_index("s")
    wid = c * NS + s

    pltpu.sync_copy(ones_hbm, ones_v)

    @pl.loop(0, TR // ZR)
    def _(i):
      pltpu.sync_copy(z_hbm, acc_sh.at[pl.ds(s * TR + i * ZR, ZR)])

    plsc.subcore_barrier()

    @pl.loop(0, NSC)
    def _(sc):
      pltpu.sync_copy(dst_hbm.at[wid, sc], dsts_v)

      @pl.loop(0, SCK)
      def _(i):
        pltpu.sync_copy(ones_v, acc_sh.at[dsts_v.at[i]], add=True)

    plsc.subcore_barrier()
    pltpu.sync_copy(acc_sh.at[pl.ds(s * TR, TR)],
                    out_hbm.at[c, pl.ds(s * TR, TR)])

  return sc_degree


_sc_deg = _make_sc_degree(8, False)
_sc_scatter128 = _make_sc_scatter(128, True)
_sc_scatter48 = _make_sc_scatter(48, False)

_BLK = 2000
_GRID = N // _BLK


def _tc_norm_body(d0_ref, d1_ref, x_ref, dinv_ref, u1_ref):
  deg = d0_ref[0] + d1_ref[0] + 1.0
  dinv = lax.rsqrt(deg)
  dinv_ref[...] = dinv
  u1_ref[...] = dinv[:, 0:1] * x_ref[...]


def _tc_norm(degs, x):
  return pl.pallas_call(
      _tc_norm_body,
      grid=(_GRID,),
      in_specs=[
          pl.BlockSpec((1, _BLK, 8), lambda i: (0, i, 0)),
          pl.BlockSpec((1, _BLK, 8), lambda i: (1, i, 0)),
          pl.BlockSpec((_BLK, 128), lambda i: (i, 0)),
      ],
      out_specs=[
          pl.BlockSpec((_BLK, 8), lambda i: (i, 0)),
          pl.BlockSpec((_BLK, 128), lambda i: (i, 0)),
      ],
      out_shape=[
          jax.ShapeDtypeStruct((N, 8), jnp.float32),
          jax.ShapeDtypeStruct((N, 128), jnp.float32),
      ],
  )(degs, degs, x)


def _tc_mid_body(a0_ref, a1_ref, u1_ref, dinv_ref, w1_ref, b1_ref, w2_ref,
                 u2_ref):
  dv = dinv_ref[:, 0:1]
  y1 = dv * (a0_ref[0] + a1_ref[0] + u1_ref[...])
  h = jnp.dot(y1, w1_ref[...], preferred_element_type=jnp.float32)
  h = jnp.maximum(h + b1_ref[...], 0.0)
  g = jnp.dot(h, w2_ref[...], preferred_element_type=jnp.float32)
  u2_ref[...] = dv * g


def _tc_mid(acc1, u1, dinv, W1, b1, W2p):
  return pl.pallas_call(
      _tc_mid_body,
      grid=(_GRID,),
      in_specs=[
          pl.BlockSpec((1, _BLK, 128), lambda i: (0, i, 0)),
          pl.BlockSpec((1, _BLK, 128), lambda i: (1, i, 0)),
          pl.BlockSpec((_BLK, 128), lambda i: (i, 0)),
          pl.BlockSpec((_BLK, 8), lambda i: (i, 0)),
          pl.BlockSpec((128, 256), lambda i: (0, 0)),
          pl.BlockSpec((1, 256), lambda i: (0, 0)),
          pl.BlockSpec((256, 48), lambda i: (0, 0)),
      ],
      out_specs=pl.BlockSpec((_BLK, 48), lambda i: (i, 0)),
      out_shape=jax.ShapeDtypeStruct((N, 48), jnp.float32),
  )(acc1, acc1, u1, dinv, W1, b1, W2p)


def _tc_out_body(a0_ref, a1_ref, u2_ref, dinv_ref, b2_ref, out_ref):
  y = dinv_ref[:, 0:1] * (a0_ref[0] + a1_ref[0] + u2_ref[...])
  y = y[:, 0:40] + b2_ref[...]
  m = jnp.max(y, axis=1, keepdims=True)
  ys = y - m
  out_ref[...] = ys - jnp.log(jnp.sum(jnp.exp(ys), axis=1, keepdims=True))


def _tc_out(acc2, u2, dinv, b2):
  return pl.pallas_call(
      _tc_out_body,
      grid=(_GRID,),
      in_specs=[
          pl.BlockSpec((1, _BLK, 48), lambda i: (0, i, 0)),
          pl.BlockSpec((1, _BLK, 48), lambda i: (1, i, 0)),
          pl.BlockSpec((_BLK, 48), lambda i: (i, 0)),
          pl.BlockSpec((_BLK, 8), lambda i: (i, 0)),
          pl.BlockSpec((1, 40), lambda i: (0, 0)),
      ],
      out_specs=pl.BlockSpec((_BLK, 40), lambda i: (i, 0)),
      out_shape=jax.ShapeDtypeStruct((N, 40), jnp.float32),
  )(acc2, acc2, u2, dinv, b2)


def kernel(x, edge_index, W1, b1, W2, b2):
  src = jnp.concatenate(
      [edge_index[0],
       jnp.zeros((EP - E,), jnp.int32)]).reshape(NW, NSC, SCK, CH)
  dst = jnp.concatenate(
      [edge_index[1],
       jnp.full((EP - E,), NPAD - 1, jnp.int32)]).reshape(NW, NSC, SCK, CH)

  ones8 = jnp.ones((CH, 8), jnp.float32)
  z8 = jnp.zeros((ZR, 8), jnp.float32)
  z128 = jnp.zeros((ZR, 128), jnp.float32)
  z48 = jnp.zeros((ZR, 48), jnp.float32)
  W2p = jnp.pad(W2, ((0, 0), (0, 8)))
  b1r = b1.reshape(1, 256)
  b2r = b2.reshape(1, 40)

  degs = _sc_deg(dst, ones8, z8)
  dinv, u1 = _tc_norm(degs, x)
  acc1 = _sc_scatter128(src, dst, u1, z128)
  u2 = _tc_mid(acc1, u1, dinv, W1, b1r, W2p)
  acc2 = _sc_scatter48(src, dst, u2, z48)
  return _tc_out(acc2, u2, dinv, b2r)
